# baseline (speedup 1.0000x reference)
import functools

import jax
import jax.numpy as jnp
from jax import lax
from jax.experimental import pallas as pl
from jax.experimental.pallas import tpu as pltpu

DIM = 1024
NUM_GATES = 16
EPS = 1e-9
CAPACITY = 160
GROUP = 2048
BATCH = 2
BLK = 256
NBLK = GROUP // BLK
LOSS_SCALE = 8.0 / (2048.0 * 2048.0)


def _softmax_top1(logits):
    m = jnp.max(logits, axis=-1, keepdims=True)
    e = jnp.exp(logits - m)
    raw = e / jnp.sum(e, axis=-1, keepdims=True)
    return raw


def _argmax_first(vals, iota):
    mx = jnp.max(vals, axis=-1, keepdims=True)
    idx = jnp.min(jnp.where(vals == mx, iota, NUM_GATES + 1), axis=-1,
                  keepdims=True)
    return mx, idx


def _stats_kernel(x_ref, w_ref, raw_ref, c1k_ref, loss_ref):
    b = pl.program_id(0)
    x = x_ref[0]
    logits = jnp.dot(x, w_ref[...], preferred_element_type=jnp.float32)
    raw = _softmax_top1(logits)
    raw_ref[0] = raw
    iota = lax.broadcasted_iota(jnp.int32, (GROUP, NUM_GATES), 1)
    _, i1 = _argmax_first(raw, iota)
    mask1 = (iota == i1).astype(jnp.float32)
    count1 = jnp.sum(mask1, axis=0, keepdims=True)
    gsum = jnp.sum(raw, axis=0, keepdims=True)
    c1k_ref[...] = jnp.minimum(count1, float(CAPACITY))[None]
    part = jnp.sum(gsum * count1, axis=1, keepdims=True) * LOSS_SCALE

    @pl.when(b == 0)
    def _():
        loss_ref[...] = part

    @pl.when(b != 0)
    def _():
        loss_ref[...] = loss_ref[...] + part


def _main_kernel(raw_ref, c1k_ref, disp_ref, comb_ref, r1_ref, r2_ref):
    k = pl.program_id(1)
    raw = raw_ref[0]
    iota = lax.broadcasted_iota(jnp.int32, (BLK, NUM_GATES), 1)
    g1, i1 = _argmax_first(raw, iota)
    mask1 = (iota == i1).astype(jnp.float32)
    wo = raw * (1.0 - mask1)
    g2, i2 = _argmax_first(wo, iota)
    mask2 = (iota == i2).astype(jnp.float32)
    den = g1 + g2 + EPS
    g1n = g1 / den
    g2n = g2 / den

    row = lax.broadcasted_iota(jnp.int32, (BLK, BLK), 0)
    col = lax.broadcasted_iota(jnp.int32, (BLK, BLK), 1)
    ltri = (row > col).astype(jnp.float32)

    zeros16 = jnp.zeros((1, NUM_GATES), jnp.float32)
    r1 = jnp.where(k == 0, zeros16, r1_ref[...])
    r2 = jnp.where(k == 0, zeros16, r2_ref[...])

    cum1 = jnp.dot(ltri, mask1, preferred_element_type=jnp.float32) + r1
    pos1 = jnp.sum(cum1 * mask1, axis=-1, keepdims=True)
    keep1 = (pos1 < float(CAPACITY)).astype(jnp.float32)
    r1_ref[...] = r1 + jnp.sum(mask1, axis=0, keepdims=True)

    cum2 = (jnp.dot(ltri, mask2, preferred_element_type=jnp.float32)
            + r2 + c1k_ref[0])
    pos2 = jnp.sum(cum2 * mask2, axis=-1, keepdims=True)
    keep2 = (pos2 < float(CAPACITY)).astype(jnp.float32)
    r2_ref[...] = r2 + jnp.sum(mask2, axis=0, keepdims=True)

    v1 = g1n * keep1
    v2 = g2n * keep2
    f1 = i1 * CAPACITY + pos1.astype(jnp.int32)
    f2 = i2 * CAPACITY + pos2.astype(jnp.int32)

    colid = lax.broadcasted_iota(jnp.int32, (BLK, NUM_GATES * CAPACITY), 1)
    comb = (jnp.where(colid == f1, v1, 0.0)
            + jnp.where(colid == f2, v2, 0.0))
    comb_ref[0] = comb
    disp_ref[0] = (comb != 0.0).astype(jnp.float32)


@jax.jit
def kernel(x, w_gating):
    raw, c1k, loss = pl.pallas_call(
        _stats_kernel,
        grid=(BATCH,),
        in_specs=[
            pl.BlockSpec((1, GROUP, DIM), lambda b: (b, 0, 0)),
            pl.BlockSpec((DIM, NUM_GATES), lambda b: (0, 0)),
        ],
        out_specs=[
            pl.BlockSpec((1, GROUP, NUM_GATES), lambda b: (b, 0, 0)),
            pl.BlockSpec((1, 1, NUM_GATES), lambda b: (b, 0, 0)),
            pl.BlockSpec((1, 1), lambda b: (0, 0)),
        ],
        out_shape=[
            jax.ShapeDtypeStruct((BATCH, GROUP, NUM_GATES), jnp.float32),
            jax.ShapeDtypeStruct((BATCH, 1, NUM_GATES), jnp.float32),
            jax.ShapeDtypeStruct((1, 1), jnp.float32),
        ],
    )(x, w_gating)

    disp, comb = pl.pallas_call(
        _main_kernel,
        grid=(BATCH, NBLK),
        in_specs=[
            pl.BlockSpec((1, BLK, NUM_GATES), lambda b, k: (b, k, 0)),
            pl.BlockSpec((1, 1, NUM_GATES), lambda b, k: (b, 0, 0)),
        ],
        out_specs=[
            pl.BlockSpec((1, BLK, NUM_GATES * CAPACITY),
                         lambda b, k: (b, k, 0)),
            pl.BlockSpec((1, BLK, NUM_GATES * CAPACITY),
                         lambda b, k: (b, k, 0)),
        ],
        out_shape=[
            jax.ShapeDtypeStruct((BATCH, GROUP, NUM_GATES * CAPACITY),
                                 jnp.float32),
            jax.ShapeDtypeStruct((BATCH, GROUP, NUM_GATES * CAPACITY),
                                 jnp.float32),
        ],
        scratch_shapes=[
            pltpu.VMEM((1, NUM_GATES), jnp.float32),
            pltpu.VMEM((1, NUM_GATES), jnp.float32),
        ],
    )(raw, c1k)

    disp = disp.reshape(BATCH, GROUP, NUM_GATES, CAPACITY)
    comb = comb.reshape(BATCH, GROUP, NUM_GATES, CAPACITY)
    return disp, comb, loss.reshape(())
